# baseline (speedup 1.0000x reference)
import functools

import jax
import jax.numpy as jnp
import numpy as np
from jax.experimental import pallas as pl
from jax.experimental.pallas import tpu as pltpu

_N_BINS = 15
_COLS = 1024
_CHUNK = 8

_POS = (0.0, 1092.2000732421875, 2184.400146484375, 3276.600341796875,
        4368.80029296875, 5461.0, 6553.20068359375, 7645.400390625,
        8737.6005859375, 9829.80078125, 10922.0, 12014.2001953125,
        13106.4013671875, 14198.6015625, 15290.80078125, 16383.0)
_RANK_LOW = tuple(int(np.floor(p)) for p in _POS)
_RANK_HIGH = tuple(int(np.ceil(p)) for p in _POS)
_HW = tuple(float(np.float32(p) - np.float32(l))
            for p, l in zip(_POS, _RANK_LOW))
_LW = tuple(float(np.float32(1.0) - np.float32(h)) for h in _HW)


def _stage_a(x_ref, lab_ref):
    lab = lab_ref[...]
    cc, n = x_ref.shape
    m8 = x_ref[0:_CHUNK, :]
    for c0 in range(_CHUNK, cc, _CHUNK):
        m8 = jnp.maximum(m8, x_ref[c0:c0 + _CHUNK, :])
    m = jnp.max(m8, axis=0, keepdims=True)
    s1a = jnp.zeros((_CHUNK, n), jnp.float32)
    s2a = jnp.zeros((_CHUNK, n), jnp.float32)
    tla = jnp.zeros((_CHUNK, n), jnp.float32)
    for c0 in range(0, cc, _CHUNK):
        x = x_ref[c0:c0 + _CHUNK, :]
        t = jnp.exp(x - m)
        row = c0 + jax.lax.broadcasted_iota(jnp.int32, x.shape, 0)
        s1a = s1a + t
        s2a = s2a + t * t
        tla = tla + jnp.where(row == lab, t, 0.0)
    s1 = jnp.sum(s1a, axis=0, keepdims=True)
    s2 = jnp.sum(s2a, axis=0, keepdims=True)
    tl = jnp.sum(tla, axis=0, keepdims=True)
    u = -jnp.log2(s2 / (s1 * s1) + 1e-12)
    e = 1.0 - tl / s1
    return u, e


def _stage_b(u, e, out_ref):
    col = jax.lax.broadcasted_iota(jnp.int32, u.shape, 1)
    rowi = jax.lax.broadcasted_iota(jnp.int32, u.shape, 0)
    s = u
    for k in range(1, 15):
        for j in range(k - 1, -1, -1):
            d = 1 << j
            if j < 7:
                a = jnp.roll(s, -d, axis=1)
                b = jnp.roll(s, d, axis=1)
                lowbit = (col & d) == 0
            else:
                dr = d >> 7
                a = jnp.roll(s, -dr, axis=0)
                b = jnp.roll(s, dr, axis=0)
                lowbit = (rowi & dr) == 0
            partner = jnp.where(lowbit, a, b)
            mn = jnp.minimum(s, partner)
            mx = jnp.maximum(s, partner)
            if k < 7:
                take_min = lowbit == ((col & (1 << k)) == 0)
            elif k < 14:
                take_min = lowbit == ((rowi & (1 << (k - 7))) == 0)
            else:
                take_min = lowbit
            s = jnp.where(take_min, mn, mx)

    edges = []
    for i in range(16):
        rl, cl = divmod(_RANK_LOW[i], 128)
        rh, ch = divmod(_RANK_HIGH[i], 128)
        edges.append(s[rl:rl + 1, cl:cl + 1] * jnp.float32(_LW[i])
                     + s[rh:rh + 1, ch:ch + 1] * jnp.float32(_HW[i]))

    total = jnp.zeros((1, 1), jnp.float32)
    for i in range(_N_BINS):
        lo_e = edges[i]
        hi_e = edges[i + 1]
        if i < _N_BINS - 1:
            mask = (u > lo_e) & (u <= hi_e)
        else:
            mask = (u >= lo_e) & (u <= hi_e)
        cntf = jnp.sum(mask.astype(jnp.float32), axis=(0, 1), keepdims=True)
        denom = jnp.maximum(cntf, 1.0)
        mu_u = jnp.sum(jnp.where(mask, u, 0.0), axis=(0, 1),
                       keepdims=True) / denom
        mu_e = jnp.sum(jnp.where(mask, e, 0.0), axis=(0, 1),
                       keepdims=True) / denom
        total = total + jnp.where(cntf > 0.0, jnp.abs(mu_u - mu_e), 0.0)
    out_ref[...] = total / jnp.float32(_N_BINS)


def _fused_kernel(x_ref, lab_ref, out_ref, u_scr, e_scr, *, grid):
    i = pl.program_id(0)
    u, e = _stage_a(x_ref, lab_ref)
    rows = _COLS // 128
    u_scr[pl.ds(i * rows, rows), :] = u.reshape(rows, 128)
    e_scr[pl.ds(i * rows, rows), :] = e.reshape(rows, 128)

    @pl.when(i == grid - 1)
    def _():
        _stage_b(u_scr[...], e_scr[...], out_ref)


def kernel(logits, labels):
    B, C = logits.shape
    xt = logits.T
    lab2 = labels.astype(jnp.int32).reshape(1, B)

    grid = B // _COLS
    out = pl.pallas_call(
        functools.partial(_fused_kernel, grid=grid),
        grid=(grid,),
        in_specs=[
            pl.BlockSpec((C, _COLS), lambda i: (0, i)),
            pl.BlockSpec((1, _COLS), lambda i: (0, i)),
        ],
        out_specs=pl.BlockSpec((1, 1), lambda i: (0, 0)),
        out_shape=jax.ShapeDtypeStruct((1, 1), jnp.float32),
        scratch_shapes=[
            pltpu.VMEM((128, 128), jnp.float32),
            pltpu.VMEM((128, 128), jnp.float32),
        ],
    )(xt, lab2)
    return out[0, 0]


# speedup vs baseline: 1.0919x; 1.0919x over previous
import functools

import jax
import jax.numpy as jnp
import numpy as np
from jax.experimental import pallas as pl
from jax.experimental.pallas import tpu as pltpu

_N_BINS = 15
_COLS = 2048
_CHUNK = 8

_POS = (0.0, 1092.2000732421875, 2184.400146484375, 3276.600341796875,
        4368.80029296875, 5461.0, 6553.20068359375, 7645.400390625,
        8737.6005859375, 9829.80078125, 10922.0, 12014.2001953125,
        13106.4013671875, 14198.6015625, 15290.80078125, 16383.0)
_RANK_LOW = tuple(int(np.floor(p)) for p in _POS)
_RANK_HIGH = tuple(int(np.ceil(p)) for p in _POS)
_HW = tuple(float(np.float32(p) - np.float32(l))
            for p, l in zip(_POS, _RANK_LOW))
_LW = tuple(float(np.float32(1.0) - np.float32(h)) for h in _HW)


def _stage_a(x_ref, lab_ref):
    lab = lab_ref[...]
    cc, n = x_ref.shape
    m8 = x_ref[0:_CHUNK, :]
    for c0 in range(_CHUNK, cc, _CHUNK):
        m8 = jnp.maximum(m8, x_ref[c0:c0 + _CHUNK, :])
    m = jnp.max(m8, axis=0, keepdims=True)
    s1a = jnp.zeros((_CHUNK, n), jnp.float32)
    s2a = jnp.zeros((_CHUNK, n), jnp.float32)
    tla = jnp.zeros((_CHUNK, n), jnp.float32)
    for c0 in range(0, cc, _CHUNK):
        x = x_ref[c0:c0 + _CHUNK, :]
        t = jnp.exp(x - m)
        row = c0 + jax.lax.broadcasted_iota(jnp.int32, x.shape, 0)
        s1a = s1a + t
        s2a = s2a + t * t
        tla = tla + jnp.where(row == lab, t, 0.0)
    s1 = jnp.sum(s1a, axis=0, keepdims=True)
    s2 = jnp.sum(s2a, axis=0, keepdims=True)
    tl = jnp.sum(tla, axis=0, keepdims=True)
    u = -jnp.log2(s2 / (s1 * s1) + 1e-12)
    e = 1.0 - tl / s1
    return u, e


def _stage_b(u, e, out_ref):
    col = jax.lax.broadcasted_iota(jnp.int32, u.shape, 1)
    rowi = jax.lax.broadcasted_iota(jnp.int32, u.shape, 0)
    s = u
    for k in range(1, 15):
        for j in range(k - 1, -1, -1):
            d = 1 << j
            if j < 7:
                a = jnp.roll(s, -d, axis=1)
                b = jnp.roll(s, d, axis=1)
                lowbit = (col & d) == 0
            else:
                dr = d >> 7
                a = jnp.roll(s, -dr, axis=0)
                b = jnp.roll(s, dr, axis=0)
                lowbit = (rowi & dr) == 0
            partner = jnp.where(lowbit, a, b)
            mn = jnp.minimum(s, partner)
            mx = jnp.maximum(s, partner)
            if k < 7:
                take_min = lowbit == ((col & (1 << k)) == 0)
            elif k < 14:
                take_min = lowbit == ((rowi & (1 << (k - 7))) == 0)
            else:
                take_min = lowbit
            s = jnp.where(take_min, mn, mx)

    edges = []
    for i in range(16):
        rl, cl = divmod(_RANK_LOW[i], 128)
        rh, ch = divmod(_RANK_HIGH[i], 128)
        edges.append(s[rl:rl + 1, cl:cl + 1] * jnp.float32(_LW[i])
                     + s[rh:rh + 1, ch:ch + 1] * jnp.float32(_HW[i]))

    total = jnp.zeros((1, 1), jnp.float32)
    for i in range(_N_BINS):
        lo_e = edges[i]
        hi_e = edges[i + 1]
        if i < _N_BINS - 1:
            mask = (u > lo_e) & (u <= hi_e)
        else:
            mask = (u >= lo_e) & (u <= hi_e)
        cntf = jnp.sum(mask.astype(jnp.float32), axis=(0, 1), keepdims=True)
        denom = jnp.maximum(cntf, 1.0)
        mu_u = jnp.sum(jnp.where(mask, u, 0.0), axis=(0, 1),
                       keepdims=True) / denom
        mu_e = jnp.sum(jnp.where(mask, e, 0.0), axis=(0, 1),
                       keepdims=True) / denom
        total = total + jnp.where(cntf > 0.0, jnp.abs(mu_u - mu_e), 0.0)
    out_ref[...] = total / jnp.float32(_N_BINS)


def _fused_kernel(x_ref, lab_ref, out_ref, u_scr, e_scr, *, grid):
    i = pl.program_id(0)
    u, e = _stage_a(x_ref, lab_ref)
    rows = _COLS // 128
    u_scr[pl.ds(i * rows, rows), :] = u.reshape(rows, 128)
    e_scr[pl.ds(i * rows, rows), :] = e.reshape(rows, 128)

    @pl.when(i == grid - 1)
    def _():
        _stage_b(u_scr[...], e_scr[...], out_ref)


def kernel(logits, labels):
    B, C = logits.shape
    xt = logits.T
    lab2 = labels.astype(jnp.int32).reshape(1, B)

    grid = B // _COLS
    out = pl.pallas_call(
        functools.partial(_fused_kernel, grid=grid),
        grid=(grid,),
        in_specs=[
            pl.BlockSpec((C, _COLS), lambda i: (0, i)),
            pl.BlockSpec((1, _COLS), lambda i: (0, i)),
        ],
        out_specs=pl.BlockSpec((1, 1), lambda i: (0, 0)),
        out_shape=jax.ShapeDtypeStruct((1, 1), jnp.float32),
        scratch_shapes=[
            pltpu.VMEM((128, 128), jnp.float32),
            pltpu.VMEM((128, 128), jnp.float32),
        ],
    )(xt, lab2)
    return out[0, 0]
